# baseline (speedup 1.0000x reference)
import jax
import jax.numpy as jnp
from jax import lax
from jax.experimental import pallas as pl
from jax.experimental.pallas import tpu as pltpu
from jax.experimental.pallas import tpu_sc as plsc

B = 16384 * 50
D = 32
GROUP = 128 // D
WROWS = 1000000 // GROUP
NC, NS = 2, 16
NW = NC * NS
BPW = B // NW
CHUNK = 160
NCHUNK = BPW // CHUNK

_mesh = plsc.VectorSubcoreMesh(core_axis_name="c", subcore_axis_name="s")


@pl.kernel(
    out_type=jax.ShapeDtypeStruct((B, D), jnp.float32),
    mesh=_mesh,
    scratch_types=[
        [pltpu.VMEM((CHUNK,), jnp.int32) for _ in range(2)],
        [pltpu.VMEM((CHUNK,), jnp.int32) for _ in range(2)],
        [pltpu.VMEM((CHUNK, 128), jnp.float32) for _ in range(2)],
        [pltpu.VMEM((CHUNK, D), jnp.float32) for _ in range(2)],
        [pltpu.SemaphoreType.DMA for _ in range(2)],
    ],
)
def _gather(w_hbm, i_hbm, o_hbm, vidx, jdx, rows, outv, sems):
    wid = lax.axis_index("s") * NC + lax.axis_index("c")
    base = wid * BPW

    def load_and_fire(c, b):
        off = base + c * CHUNK
        pltpu.sync_copy(i_hbm.at[pl.ds(off, CHUNK)], vidx[b])

        @pl.loop(0, CHUNK, step=16)
        def _(p):
            jdx[b][pl.ds(p, 16)] = lax.shift_right_logical(
                vidx[b][pl.ds(p, 16)], 2
            )

        pltpu.async_copy(w_hbm.at[jdx[b]], rows[b], sems[b])

    def drain(c, b):
        pltpu.make_async_copy(w_hbm.at[jdx[b]], rows[b], sems[b]).wait()

        @pl.loop(0, CHUNK, step=16)
        def _(p):
            iv = vidx[b][pl.ds(p, 16)]
            for k in range(16):
                r32 = (iv[k] & 3) * D
                outv[b][p + k, pl.ds(0, 16)] = rows[b][p + k, pl.ds(r32, 16)]
                outv[b][p + k, pl.ds(16, 16)] = rows[b][p + k, pl.ds(r32 + 16, 16)]

        pltpu.sync_copy(outv[b], o_hbm.at[pl.ds(base + c * CHUNK, CHUNK)])

    load_and_fire(0, 0)

    @pl.loop(0, NCHUNK, step=2)
    def _(c):
        load_and_fire(c + 1, 1)
        drain(c, 0)

        @pl.when(c + 2 < NCHUNK)
        def _():
            load_and_fire(c + 2, 0)

        drain(c + 1, 1)


@jax.jit
def kernel(weight, input_index):
    w128 = weight.reshape(WROWS, 128)
    flat_idx = input_index.reshape(B)
    out = _gather(w128, flat_idx)
    return out.reshape(*input_index.shape, D)


# speedup vs baseline: 1.2948x; 1.2948x over previous
import jax
import jax.numpy as jnp
from jax import lax
from jax.experimental import pallas as pl
from jax.experimental.pallas import tpu as pltpu
from jax.experimental.pallas import tpu_sc as plsc

NROW = 16384
NCOL = 50
D = 32
GROUP = 4
WBLKS = 1000000 // GROUP
NC, NS = 2, 16
NW = NC * NS
RPW = NROW // NW

_mesh = plsc.VectorSubcoreMesh(core_axis_name="c", subcore_axis_name="s")


@pl.kernel(
    out_type=jax.ShapeDtypeStruct((NROW, NCOL, D), jnp.float32),
    mesh=_mesh,
    scratch_types=[
        [pltpu.VMEM((NCOL,), jnp.int32) for _ in range(2)],
        [pltpu.VMEM((NCOL,), jnp.int32) for _ in range(2)],
        [pltpu.VMEM((NCOL, GROUP * D), jnp.float32) for _ in range(2)],
        [pltpu.VMEM((NCOL, D), jnp.float32) for _ in range(2)],
        [pltpu.SemaphoreType.DMA for _ in range(2)],
    ],
)
def _gather(w_hbm, i_hbm, o_hbm, idxv, jdxv, rows, outv, sems):
    wid = lax.axis_index("s") * NC + lax.axis_index("c")
    base = wid * RPW

    def fire(i, b):
        pltpu.sync_copy(i_hbm.at[i], idxv[b])
        for q in (0, 16, 32, 34):
            jdxv[b][pl.ds(q, 16)] = lax.shift_right_logical(
                idxv[b][pl.ds(q, 16)], 2
            )
        pltpu.async_copy(w_hbm.at[jdxv[b]], rows[b], sems[b])

    def drain(i, b):
        pltpu.make_async_copy(w_hbm.at[jdxv[b]], rows[b], sems[b]).wait()
        for q in (0, 16, 32, 34):
            iv = idxv[b][pl.ds(q, 16)]
            lo = 14 if q == 34 else 0
            for k in range(lo, min(16, NCOL - q)):
                r32 = (iv[k] & (GROUP - 1)) * D
                outv[b][q + k, pl.ds(0, 16)] = rows[b][q + k, pl.ds(r32, 16)]
                outv[b][q + k, pl.ds(16, 16)] = rows[b][q + k, pl.ds(r32 + 16, 16)]
        pltpu.sync_copy(outv[b], o_hbm.at[i])

    fire(base, 0)

    @pl.loop(0, RPW, step=2)
    def _(c):
        fire(base + c + 1, 1)
        drain(base + c, 0)

        @pl.when(c + 2 < RPW)
        def _():
            fire(base + c + 2, 0)

        drain(base + c + 1, 1)


@jax.jit
def kernel(weight, input_index):
    w3 = weight.reshape(WBLKS, GROUP * D)
    return _gather(w3, input_index)


# speedup vs baseline: 1.4464x; 1.1170x over previous
import jax
import jax.numpy as jnp
from jax import lax
from jax.experimental import pallas as pl
from jax.experimental.pallas import tpu as pltpu
from jax.experimental.pallas import tpu_sc as plsc

NROW = 16384
NCOL = 50
D = 32
GROUP = 4
WBLKS = 1000000 // GROUP
NC, NS = 2, 16
NW = NC * NS
RPW = NROW // NW
CHUNKI = 2
NIDX = CHUNKI * NCOL
NSTEP = RPW // CHUNKI

_mesh = plsc.VectorSubcoreMesh(core_axis_name="c", subcore_axis_name="s")

_GROUPS = [(0, 0), (16, 0), (32, 0), (34, 14)]


@pl.kernel(
    out_type=jax.ShapeDtypeStruct((NROW, NCOL, D), jnp.float32),
    mesh=_mesh,
    scratch_types=[
        [pltpu.VMEM((CHUNKI, NCOL), jnp.int32) for _ in range(2)],
        [pltpu.VMEM((NIDX,), jnp.int32) for _ in range(2)],
        [pltpu.VMEM((NIDX, 128), jnp.float32) for _ in range(2)],
        [pltpu.VMEM((CHUNKI, NCOL, D), jnp.float32) for _ in range(2)],
        [pltpu.SemaphoreType.DMA for _ in range(2)],
    ],
)
def _gather(w_hbm, i_hbm, o_hbm, idxv, jdxv, rows, outv, sems):
    wid = lax.axis_index("s") * NC + lax.axis_index("c")
    base = wid * RPW

    def fire(i, b):
        pltpu.sync_copy(i_hbm.at[pl.ds(i, CHUNKI)], idxv[b])
        for ii in range(CHUNKI):
            for q, _ in _GROUPS:
                jdxv[b][pl.ds(ii * NCOL + q, 16)] = lax.shift_right_logical(
                    idxv[b][ii, pl.ds(q, 16)], 2
                )
        pltpu.async_copy(w_hbm.at[jdxv[b]], rows[b], sems[b])

    def drain(i, b):
        pltpu.make_async_copy(w_hbm.at[jdxv[b]], rows[b], sems[b]).wait()
        for ii in range(CHUNKI):
            for q, lo in _GROUPS:
                iv = idxv[b][ii, pl.ds(q, 16)]
                for k in range(lo, min(16, NCOL - q)):
                    r32 = (iv[k] & (GROUP - 1)) * D
                    outv[b][ii, q + k, pl.ds(0, 16)] = rows[b][
                        ii * NCOL + q + k, pl.ds(r32, 16)
                    ]
                    outv[b][ii, q + k, pl.ds(16, 16)] = rows[b][
                        ii * NCOL + q + k, pl.ds(r32 + 16, 16)
                    ]
        pltpu.sync_copy(outv[b], o_hbm.at[pl.ds(i, CHUNKI)])

    fire(base, 0)

    @pl.loop(0, NSTEP, step=2)
    def _(c):
        fire(base + (c + 1) * CHUNKI, 1)
        drain(base + c * CHUNKI, 0)

        @pl.when(c + 2 < NSTEP)
        def _():
            fire(base + (c + 2) * CHUNKI, 0)

        drain(base + (c + 1) * CHUNKI, 1)


@jax.jit
def kernel(weight, input_index):
    w128 = weight.reshape(WBLKS, GROUP * D)
    return _gather(w128, input_index)


# speedup vs baseline: 1.4586x; 1.0085x over previous
import jax
import jax.numpy as jnp
from jax import lax
from jax.experimental import pallas as pl
from jax.experimental.pallas import tpu as pltpu
from jax.experimental.pallas import tpu_sc as plsc

NROW = 16384
NCOL = 50
D = 32
GROUP = 4
WBLKS = 1000000 // GROUP
NC, NS = 2, 16
NW = NC * NS
RPW = NROW // NW
CHUNKI = 4
NIDX = CHUNKI * NCOL
NSTEP = RPW // CHUNKI

_mesh = plsc.VectorSubcoreMesh(core_axis_name="c", subcore_axis_name="s")

_GROUPS = [(0, 0), (16, 0), (32, 0), (34, 14)]


@pl.kernel(
    out_type=jax.ShapeDtypeStruct((NROW, NCOL, D), jnp.float32),
    mesh=_mesh,
    scratch_types=[
        [pltpu.VMEM((CHUNKI, NCOL), jnp.int32) for _ in range(2)],
        [pltpu.VMEM((NIDX,), jnp.int32) for _ in range(2)],
        [pltpu.VMEM((NIDX, 128), jnp.float32) for _ in range(2)],
        [pltpu.VMEM((CHUNKI, NCOL, D), jnp.float32) for _ in range(2)],
        [pltpu.SemaphoreType.DMA for _ in range(2)],
    ],
)
def _gather(w_hbm, i_hbm, o_hbm, idxv, jdxv, rows, outv, sems):
    wid = lax.axis_index("s") * NC + lax.axis_index("c")
    base = wid * RPW

    def fire(i, b):
        pltpu.sync_copy(i_hbm.at[pl.ds(i, CHUNKI)], idxv[b])
        for ii in range(CHUNKI):
            for q, _ in _GROUPS:
                jdxv[b][pl.ds(ii * NCOL + q, 16)] = lax.shift_right_logical(
                    idxv[b][ii, pl.ds(q, 16)], 2
                )
        pltpu.async_copy(w_hbm.at[jdxv[b]], rows[b], sems[b])

    def drain(i, b):
        pltpu.make_async_copy(w_hbm.at[jdxv[b]], rows[b], sems[b]).wait()
        for ii in range(CHUNKI):
            for q, lo in _GROUPS:
                iv = idxv[b][ii, pl.ds(q, 16)]
                for k in range(lo, min(16, NCOL - q)):
                    r32 = (iv[k] & (GROUP - 1)) * D
                    outv[b][ii, q + k, pl.ds(0, 16)] = rows[b][
                        ii * NCOL + q + k, pl.ds(r32, 16)
                    ]
                    outv[b][ii, q + k, pl.ds(16, 16)] = rows[b][
                        ii * NCOL + q + k, pl.ds(r32 + 16, 16)
                    ]
        pltpu.sync_copy(outv[b], o_hbm.at[pl.ds(i, CHUNKI)])

    fire(base, 0)

    @pl.loop(0, NSTEP, step=2)
    def _(c):
        fire(base + (c + 1) * CHUNKI, 1)
        drain(base + c * CHUNKI, 0)

        @pl.when(c + 2 < NSTEP)
        def _():
            fire(base + (c + 2) * CHUNKI, 0)

        drain(base + (c + 1) * CHUNKI, 1)


@jax.jit
def kernel(weight, input_index):
    w128 = weight.reshape(WBLKS, GROUP * D)
    return _gather(w128, input_index)
